# baseline (speedup 1.0000x reference)
import functools
import math

import jax
import jax.numpy as jnp
from jax import lax
from jax.experimental import pallas as pl
from jax.experimental.pallas import tpu as pltpu
from jax.experimental.pallas import tpu_sc as plsc

EMB = 64
SCALE = math.sqrt(EMB)

NC = 2
NS = 16
NW = NC * NS

CHUNK = 128
NBUF = 4
LANE = 16

ET = EMB // 8


def _scale_body(x_ref, o_ref):
    o_ref[...] = x_ref[...] * SCALE


def _scale_table(table):
    v, d = table.shape
    n = v * d
    cols = 128
    rows = n // cols
    block_rows = rows // 10
    t2 = table.reshape(rows, cols)
    scaled = pl.pallas_call(
        _scale_body,
        out_shape=jax.ShapeDtypeStruct((rows, cols), jnp.float32),
        grid=(10,),
        in_specs=[pl.BlockSpec((block_rows, cols), lambda i: (i, 0))],
        out_specs=pl.BlockSpec((block_rows, cols), lambda i: (i, 0)),
    )(t2)
    return scaled.reshape(v, d)


@functools.partial(jax.jit, static_argnames=("n_chunks", "n_pos"))
def _sc_gather(idx3, table_scaled, *, n_chunks, n_pos):
    assert n_chunks % NBUF == 0 and n_chunks >= 2 * NBUF
    b_total = NW * n_chunks * CHUNK
    n_seq = b_total // n_pos
    chunks_per_pos = n_seq // CHUNK
    mesh = plsc.VectorSubcoreMesh(core_axis_name="c", subcore_axis_name="s")

    @functools.partial(
        pl.kernel,
        out_type=jax.ShapeDtypeStruct(
            (n_pos, ET, chunks_per_pos, 8, CHUNK), jnp.float32),
        mesh=mesh,
        scratch_types=(
            [pltpu.VMEM((n_chunks, CHUNK), jnp.int32),
             pltpu.VMEM((NBUF, CHUNK, EMB), jnp.float32),
             pltpu.VMEM((NBUF, ET, 8, CHUNK), jnp.float32)]
            + [pltpu.SemaphoreType.DMA] * (2 * NBUF)
        ),
        compiler_params=pltpu.CompilerParams(use_tc_tiling_on_sc=False,
                                             needs_layout_passes=False),
    )
    def k(idx_hbm, tab_hbm, out_hbm, idx_v, rows_v, tr_v, *sems):
        gsems, osems = sems[:NBUF], sems[NBUF:]
        wid = lax.axis_index("s") * NC + lax.axis_index("c")
        pltpu.sync_copy(idx_hbm.at[wid], idx_v)

        iota = lax.iota(jnp.int32, LANE)
        row_idx = [iota + lb * LANE for lb in range(CHUNK // LANE)]

        def start_gather(j, b):
            pltpu.async_copy(tab_hbm.at[idx_v.at[j]], rows_v.at[b], gsems[b])

        def wait_gather(b):
            pltpu.make_async_copy(
                tab_hbm.at[idx_v.at[0]], rows_v.at[b], gsems[b]).wait()

        def start_out(j, b):
            c = wid * n_chunks + j
            p = c // chunks_per_pos
            st = c % chunks_per_pos
            pltpu.async_copy(
                tr_v.at[b], out_hbm.at[p, :, st, :, :], osems[b])

        def wait_out(b):
            pltpu.make_async_copy(
                tr_v.at[b], out_hbm.at[0, :, 0, :, :], osems[b]).wait()

        def transpose_chunk(b):
            rows = rows_v.at[b]
            tr = tr_v.at[b]
            for et in range(ET):
                for es in range(8):
                    col = jnp.full((LANE,), et * 8 + es, jnp.int32)
                    for lb in range(CHUNK // LANE):
                        val = plsc.load_gather(rows, [row_idx[lb], col])
                        tr[et, es, pl.ds(lb * LANE, LANE)] = val

        for b in range(NBUF):
            start_gather(b, b)

        @pl.loop(0, n_chunks, step=NBUF)
        def _blk(j0):
            for i in range(NBUF):
                j = j0 + i
                wait_gather(i)

                @pl.when(j >= NBUF)
                def _w():
                    wait_out(i)

                transpose_chunk(i)
                start_out(j, i)

                @pl.when(j + NBUF < n_chunks)
                def _g():
                    start_gather(j + NBUF, i)

        for b in range(NBUF):
            wait_out(b)

    return k(idx3, table_scaled)


def kernel(tokens, table):
    s, p = tokens.shape
    b_total = s * p
    n_chunks = b_total // (NW * CHUNK)
    idx3 = tokens.T.reshape(NW, n_chunks, CHUNK).astype(jnp.int32)
    table_scaled = _scale_table(table)
    out5 = _sc_gather(idx3, table_scaled, n_chunks=n_chunks, n_pos=p)
    return out5.transpose(2, 4, 0, 1, 3).reshape(s, p, EMB)
